# baseline (speedup 1.0000x reference)
import functools

import jax
import jax.numpy as jnp
from jax import lax
from jax.experimental import pallas as pl
from jax.experimental.pallas import tpu as pltpu
from jax.experimental.pallas import tpu_sc as plsc

N = 10000
E = 320000
D = 128
G = 64
NC = 2
NS = 16
NW = NC * NS
NP = 10240
RT = NP // NS
K = 96
EW = E // NW
BT = 216
TB = NS * BT
TBP = 3616
B0 = 184
B1 = BT - B0
EPAD = TB * K
R = 1280
NB = NP // R


def _mesh():
    return plsc.VectorSubcoreMesh(
        core_axis_name="c", subcore_axis_name="s", num_cores=NC, num_subcores=NS
    )


@functools.partial(
    pl.kernel,
    out_type=jax.ShapeDtypeStruct((NC, NP), jnp.float32),
    mesh=_mesh(),
    scratch_types=[
        pltpu.VMEM((EW,), jnp.int32),
        pltpu.VMEM((NP,), jnp.float32),
        pltpu.VMEM((RT,), jnp.float32),
        pltpu.VMEM((RT,), jnp.float32),
        pltpu.VMEM_SHARED((NS, NP), jnp.float32),
    ],
    compiler_params=pltpu.CompilerParams(needs_layout_passes=False),
)
def _deg_kernel(dst_hbm, out_hbm, dst_v, hist_v, col_v, tmp_v, stage_sh):
    c = lax.axis_index("c")
    s = lax.axis_index("s")
    w = c * NS + s
    pltpu.sync_copy(dst_hbm.at[pl.ds(w * EW, EW)], dst_v)
    zero16 = jnp.zeros((16,), jnp.float32)

    def zb(i, _):
        hist_v[pl.ds(i * 16, 16)] = zero16
        return 0

    lax.fori_loop(0, NP // 16, zb, 0, unroll=4)
    one16 = jnp.ones((16,), jnp.float32)

    def sb(i, _):
        plsc.addupdate_scatter(hist_v, [dst_v[pl.ds(i * 16, 16)]], one16)
        return 0

    lax.fori_loop(0, EW // 16, sb, 0, unroll=4)
    pltpu.sync_copy(hist_v, stage_sh.at[s])
    plsc.subcore_barrier()
    base = s * RT
    pltpu.sync_copy(stage_sh.at[0, pl.ds(base, RT)], col_v)
    for t in range(1, NS):
        pltpu.sync_copy(stage_sh.at[t, pl.ds(base, RT)], tmp_v)

        def ab(j, _):
            col_v[pl.ds(j * 16, 16)] = col_v[pl.ds(j * 16, 16)] + tmp_v[pl.ds(j * 16, 16)]
            return 0

        lax.fori_loop(0, RT // 16, ab, 0, unroll=4)
    pltpu.sync_copy(col_v, out_hbm.at[c, pl.ds(base, RT)])


@functools.partial(
    pl.kernel,
    out_type=jax.ShapeDtypeStruct((NC, NP, D), jnp.float32),
    mesh=_mesh(),
    scratch_types=[
        pltpu.VMEM((B0, K), jnp.int32),
        [pltpu.VMEM((K,), jnp.int32) for _ in range(2)],
        [pltpu.VMEM((K,), jnp.int32) for _ in range(2)],
        [pltpu.VMEM((K, D), jnp.float32) for _ in range(2)],
        pltpu.VMEM_SHARED((NP, D), jnp.float32),
        pltpu.SemaphoreType.DMA,
        [pltpu.SemaphoreType.DMA for _ in range(2)],
        [pltpu.SemaphoreType.DMA for _ in range(2)],
    ],
    compiler_params=pltpu.CompilerParams(needs_layout_passes=False),
)
def _seg_kernel(comb_hbm, y_hbm, out_hbm,
                comb_v, srcb, dstb, rows, acc_sh, csem, gsems, ssems):
    c = lax.axis_index("c")
    s = lax.axis_index("s")
    base = s * RT
    start = s * BT + c * B0
    nb = jnp.where(c == 0, B0, B1)
    pltpu.async_copy(comb_hbm.at[pl.ds(start, B0)], comb_v, csem)
    zero16 = jnp.zeros((16,), jnp.float32)

    def zb(i, _):
        for j in range(D // 16):
            rows[0][i, pl.ds(j * 16, 16)] = zero16
        return 0

    lax.fori_loop(0, K, zb, 0, unroll=4)
    for j in range(RT // K):
        pltpu.sync_copy(rows[0], acc_sh.at[pl.ds(base + j * K, K)])
    rem = RT - (RT // K) * K
    if rem:
        pltpu.sync_copy(
            rows[0].at[pl.ds(0, rem)], acc_sh.at[pl.ds(base + (RT // K) * K, rem)]
        )
    plsc.subcore_barrier()
    pltpu.make_async_copy(comb_hbm.at[pl.ds(start, B0)], comb_v, csem).wait()

    def unpack(r, slot):
        for g in range(K // 16):
            vv = comb_v[r, pl.ds(g * 16, 16)]
            srcb[slot][pl.ds(g * 16, 16)] = jnp.bitwise_and(vv, 0xFFFF)
            dstb[slot][pl.ds(g * 16, 16)] = jax.lax.shift_right_logical(vv, 16)

    unpack(0, 0)
    pltpu.async_copy(y_hbm.at[srcb[0]], rows[0], gsems[0])

    def eb(j, _):
        for b in range(2):
            i = j * 2 + b
            pltpu.make_async_copy(y_hbm.at[srcb[b]], rows[b], gsems[b]).wait()

            @pl.when((i + 1 < nb) & (i >= 1))
            def _w():
                pltpu.make_async_copy(
                    rows[1 - b], acc_sh.at[dstb[1 - b]], ssems[1 - b]
                ).wait()

            @pl.when(i + 1 < nb)
            def _g():
                unpack(i + 1, 1 - b)
                pltpu.async_copy(y_hbm.at[srcb[1 - b]], rows[1 - b], gsems[1 - b])

            pltpu.async_copy(rows[b], acc_sh.at[dstb[b]], ssems[b], add=True)
        return 0

    lax.fori_loop(0, jnp.where(c == 0, B0 // 2, B1 // 2), eb, 0)
    for b in range(2):
        pltpu.make_async_copy(rows[b], acc_sh.at[dstb[b]], ssems[b]).wait()
    plsc.subcore_barrier()
    pltpu.sync_copy(acc_sh.at[pl.ds(base, RT)], out_hbm.at[c, pl.ds(base, RT)])


def _tc1_body(dega, degb, x, w1, y, disb):
    i = pl.program_id(0)
    rows = lax.broadcasted_iota(jnp.int32, (R, 1), 0) + i * R
    m = (rows < N).astype(jnp.float32)
    dis = m * lax.rsqrt(dega[...] + degb[...] + 1.0)
    h = jnp.dot(x[...], w1[...], preferred_element_type=jnp.float32)
    y[...] = h * dis
    disb[...] = jnp.broadcast_to(dis, (R, D))


_tc1 = pl.pallas_call(
    _tc1_body,
    grid=(NB,),
    in_specs=[
        pl.BlockSpec((R, 1), lambda i: (i, 0)),
        pl.BlockSpec((R, 1), lambda i: (i, 0)),
        pl.BlockSpec((R, D), lambda i: (i, 0)),
        pl.BlockSpec((D, D), lambda i: (0, 0)),
    ],
    out_specs=[pl.BlockSpec((R, D), lambda i: (i, 0))] * 2,
    out_shape=[jax.ShapeDtypeStruct((NP, D), jnp.float32)] * 2,
)


def _tc2_body(sa, sb, y1, disb, b1, w2, y2):
    dis = disb[...]
    pre = dis * (sa[...] + sb[...] + y1[...]) + b1[...]
    a = jnp.where(pre > 0, pre, jnp.exp(pre) - 1.0)
    y2[...] = jnp.dot(a, w2[...], preferred_element_type=jnp.float32) * dis


_tc2 = pl.pallas_call(
    _tc2_body,
    grid=(NB,),
    in_specs=[
        pl.BlockSpec((R, D), lambda i: (i, 0)),
        pl.BlockSpec((R, D), lambda i: (i, 0)),
        pl.BlockSpec((R, D), lambda i: (i, 0)),
        pl.BlockSpec((R, D), lambda i: (i, 0)),
        pl.BlockSpec((1, D), lambda i: (0, 0)),
        pl.BlockSpec((D, D), lambda i: (0, 0)),
    ],
    out_specs=pl.BlockSpec((R, D), lambda i: (i, 0)),
    out_shape=jax.ShapeDtypeStruct((NP, D), jnp.float32),
)


def _tc3_body(sa, sb, y2, disb, b2, batr, stats, f1wa, f1wb, f1b, f2w, f2b,
              out, pool_acc, cnt_acc):
    i = pl.program_id(0)
    dis = disb[...]
    out2 = dis * (sa[...] + sb[...] + y2[...]) + b2[...]
    bat = jnp.reshape(batr[...], (1, R))
    onehot = (
        lax.broadcasted_iota(jnp.int32, (G, R), 0) == jnp.broadcast_to(bat, (G, R))
    ).astype(jnp.float32)
    psum = jnp.dot(onehot, out2, preferred_element_type=jnp.float32)
    csum = jnp.sum(onehot, axis=1, keepdims=True)

    @pl.when(i == 0)
    def _init():
        pool_acc[...] = psum
        cnt_acc[...] = csum

    @pl.when(i > 0)
    def _acc():
        pool_acc[...] += psum
        cnt_acc[...] += csum

    @pl.when(i == NB - 1)
    def _fin():
        pooled = pool_acc[...] / jnp.maximum(cnt_acc[...], 1.0)
        z = (
            jnp.dot(pooled, f1wa[...], preferred_element_type=jnp.float32)
            + jnp.dot(stats[...], f1wb[...], preferred_element_type=jnp.float32)
            + f1b[...]
        )
        z = jnp.maximum(z, 0.0)
        z = jnp.dot(z, f2w[...], preferred_element_type=jnp.float32) + f2b[...]
        m = jnp.max(z, axis=1, keepdims=True)
        e = jnp.exp(z - m)
        out[...] = z - m - jnp.log(jnp.sum(e, axis=1, keepdims=True))


_tc3 = pl.pallas_call(
    _tc3_body,
    grid=(NB,),
    in_specs=[
        pl.BlockSpec((R, D), lambda i: (i, 0)),
        pl.BlockSpec((R, D), lambda i: (i, 0)),
        pl.BlockSpec((R, D), lambda i: (i, 0)),
        pl.BlockSpec((R, D), lambda i: (i, 0)),
        pl.BlockSpec((1, D), lambda i: (0, 0)),
        pl.BlockSpec((1, 1, R), lambda i: (i, 0, 0)),
        pl.BlockSpec((G, 10), lambda i: (0, 0)),
        pl.BlockSpec((D, 20), lambda i: (0, 0)),
        pl.BlockSpec((10, 20), lambda i: (0, 0)),
        pl.BlockSpec((1, 20), lambda i: (0, 0)),
        pl.BlockSpec((20, 5), lambda i: (0, 0)),
        pl.BlockSpec((1, 5), lambda i: (0, 0)),
    ],
    out_specs=pl.BlockSpec((G, 5), lambda i: (0, 0)),
    out_shape=jax.ShapeDtypeStruct((G, 5), jnp.float32),
    scratch_shapes=[
        pltpu.VMEM((G, D), jnp.float32),
        pltpu.VMEM((G, 1), jnp.float32),
    ],
)


def kernel(x, edge_index, batch, eig, stats, W1, b1, W2, b2,
           fc1_W, fc1_b, fc2_W, fc2_b):
    del eig
    src = edge_index[0]
    dst = edge_index[1]
    pad = jnp.full((EPAD - E,), N, jnp.int32)
    srcp = jnp.concatenate([src, pad])
    dstp = jnp.concatenate([dst, pad])
    comb = jnp.bitwise_or(srcp, dstp << 16).reshape(TB, K)
    comb = jnp.zeros((TBP, K), jnp.int32).at[:TB].set(comb)
    xp = jnp.zeros((NP, D), jnp.float32).at[:N].set(x)

    deg2 = _deg_kernel(dst)
    deg_a = deg2[0].reshape(NP, 1)
    deg_b = deg2[1].reshape(NP, 1)
    y1, dis_b = _tc1(deg_a, deg_b, xp, W1)
    seg1 = _seg_kernel(comb, y1)
    y2 = _tc2(seg1[0], seg1[1], y1, dis_b, b1.reshape(1, D), W2)
    seg2 = _seg_kernel(comb, y2)
    batp = jnp.full((NP,), G, jnp.int32).at[:N].set(batch).reshape(NB, 1, R)
    return _tc3(seg2[0], seg2[1], y2, dis_b, b2.reshape(1, D),
                batp, stats, fc1_W[:D], fc1_W[D:], fc1_b.reshape(1, 20),
                fc2_W, fc2_b.reshape(1, 5))


# speedup vs baseline: 1.0002x; 1.0002x over previous
import functools

import jax
import jax.numpy as jnp
from jax import lax
from jax.experimental import pallas as pl
from jax.experimental.pallas import tpu as pltpu
from jax.experimental.pallas import tpu_sc as plsc

N = 10000
E = 320000
D = 128
G = 64
NC = 2
NS = 16
NW = NC * NS
NP = 10240
RT = NP // NS
K = 96
EW = E // NW
BT = 216
TB = NS * BT
TBP = 3616
B0 = 184
B1 = BT - B0
EPAD = TB * K
R = 1280
NB = NP // R


def _mesh():
    return plsc.VectorSubcoreMesh(
        core_axis_name="c", subcore_axis_name="s", num_cores=NC, num_subcores=NS
    )


@functools.partial(
    pl.kernel,
    out_type=jax.ShapeDtypeStruct((NC, NP), jnp.float32),
    mesh=_mesh(),
    scratch_types=[
        pltpu.VMEM((EW,), jnp.int32),
        pltpu.VMEM((NP,), jnp.float32),
        pltpu.VMEM((RT,), jnp.float32),
        pltpu.VMEM((RT,), jnp.float32),
        pltpu.VMEM_SHARED((NS, NP), jnp.float32),
    ],
    compiler_params=pltpu.CompilerParams(needs_layout_passes=False),
)
def _deg_kernel(dst_hbm, out_hbm, dst_v, hist_v, col_v, tmp_v, stage_sh):
    c = lax.axis_index("c")
    s = lax.axis_index("s")
    w = c * NS + s
    pltpu.sync_copy(dst_hbm.at[pl.ds(w * EW, EW)], dst_v)
    zero16 = jnp.zeros((16,), jnp.float32)

    def zb(i, _):
        hist_v[pl.ds(i * 16, 16)] = zero16
        return 0

    lax.fori_loop(0, NP // 16, zb, 0, unroll=4)
    one16 = jnp.ones((16,), jnp.float32)

    def sb(i, _):
        plsc.addupdate_scatter(hist_v, [dst_v[pl.ds(i * 16, 16)]], one16)
        return 0

    lax.fori_loop(0, EW // 16, sb, 0, unroll=4)
    pltpu.sync_copy(hist_v, stage_sh.at[s])
    plsc.subcore_barrier()
    base = s * RT
    pltpu.sync_copy(stage_sh.at[0, pl.ds(base, RT)], col_v)
    for t in range(1, NS):
        pltpu.sync_copy(stage_sh.at[t, pl.ds(base, RT)], tmp_v)

        def ab(j, _):
            col_v[pl.ds(j * 16, 16)] = col_v[pl.ds(j * 16, 16)] + tmp_v[pl.ds(j * 16, 16)]
            return 0

        lax.fori_loop(0, RT // 16, ab, 0, unroll=4)
    pltpu.sync_copy(col_v, out_hbm.at[c, pl.ds(base, RT)])


@functools.partial(
    pl.kernel,
    out_type=jax.ShapeDtypeStruct((NC, NP, D), jnp.float32),
    mesh=_mesh(),
    scratch_types=[
        pltpu.VMEM((B0, K), jnp.int32),
        [pltpu.VMEM((K,), jnp.int32) for _ in range(2)],
        [pltpu.VMEM((K,), jnp.int32) for _ in range(2)],
        [pltpu.VMEM((K, D), jnp.float32) for _ in range(2)],
        pltpu.VMEM_SHARED((NP, D), jnp.float32),
        pltpu.SemaphoreType.DMA,
        [pltpu.SemaphoreType.DMA for _ in range(2)],
        [pltpu.SemaphoreType.DMA for _ in range(2)],
    ],
    compiler_params=pltpu.CompilerParams(needs_layout_passes=False),
)
def _seg_kernel(comb_hbm, y_hbm, out_hbm,
                comb_v, srcb, dstb, rows, acc_sh, csem, gsems, ssems):
    c = lax.axis_index("c")
    s = lax.axis_index("s")
    base = s * RT
    start = s * BT + c * B0
    nb = jnp.where(c == 0, B0, B1)
    pltpu.async_copy(comb_hbm.at[pl.ds(start, B0)], comb_v, csem)
    zero16 = jnp.zeros((16,), jnp.float32)

    def zb(i, _):
        for j in range(D // 16):
            rows[0][i, pl.ds(j * 16, 16)] = zero16
        return 0

    lax.fori_loop(0, K, zb, 0, unroll=4)
    for j in range(RT // K):
        pltpu.sync_copy(rows[0], acc_sh.at[pl.ds(base + j * K, K)])
    rem = RT - (RT // K) * K
    if rem:
        pltpu.sync_copy(
            rows[0].at[pl.ds(0, rem)], acc_sh.at[pl.ds(base + (RT // K) * K, rem)]
        )
    plsc.subcore_barrier()
    pltpu.make_async_copy(comb_hbm.at[pl.ds(start, B0)], comb_v, csem).wait()

    def unpack(r, slot):
        for g in range(K // 16):
            vv = comb_v[r, pl.ds(g * 16, 16)]
            srcb[slot][pl.ds(g * 16, 16)] = jnp.bitwise_and(vv, 0xFFFF)
            dstb[slot][pl.ds(g * 16, 16)] = jax.lax.shift_right_logical(vv, 16)

    unpack(0, 0)
    pltpu.async_copy(y_hbm.at[srcb[0]], rows[0], gsems[0])

    def run(nbs):
        def eb(j, _):
            for b in range(2):
                i = j * 2 + b
                pltpu.make_async_copy(y_hbm.at[srcb[b]], rows[b], gsems[b]).wait()

                @pl.when((i + 1 < nbs) & (i >= 1))
                def _w():
                    pltpu.make_async_copy(
                        rows[1 - b], acc_sh.at[dstb[1 - b]], ssems[1 - b]
                    ).wait()

                @pl.when(i + 1 < nbs)
                def _g():
                    unpack(i + 1, 1 - b)
                    pltpu.async_copy(y_hbm.at[srcb[1 - b]], rows[1 - b], gsems[1 - b])

                pltpu.async_copy(rows[b], acc_sh.at[dstb[b]], ssems[b], add=True)
            return 0

        lax.fori_loop(0, nbs // 2, eb, 0)

    @pl.when(c == 0)
    def _run0():
        run(B0)

    @pl.when(c == 1)
    def _run1():
        run(B1)

    for b in range(2):
        pltpu.make_async_copy(rows[b], acc_sh.at[dstb[b]], ssems[b]).wait()
    plsc.subcore_barrier()
    pltpu.sync_copy(acc_sh.at[pl.ds(base, RT)], out_hbm.at[c, pl.ds(base, RT)])


def _tc1_body(dega, degb, x, w1, y, disb):
    i = pl.program_id(0)
    rows = lax.broadcasted_iota(jnp.int32, (R, 1), 0) + i * R
    m = (rows < N).astype(jnp.float32)
    dis = m * lax.rsqrt(dega[...] + degb[...] + 1.0)
    h = jnp.dot(x[...], w1[...], preferred_element_type=jnp.float32)
    y[...] = h * dis
    disb[...] = jnp.broadcast_to(dis, (R, D))


_tc1 = pl.pallas_call(
    _tc1_body,
    grid=(NB,),
    in_specs=[
        pl.BlockSpec((R, 1), lambda i: (i, 0)),
        pl.BlockSpec((R, 1), lambda i: (i, 0)),
        pl.BlockSpec((R, D), lambda i: (i, 0)),
        pl.BlockSpec((D, D), lambda i: (0, 0)),
    ],
    out_specs=[pl.BlockSpec((R, D), lambda i: (i, 0))] * 2,
    out_shape=[jax.ShapeDtypeStruct((NP, D), jnp.float32)] * 2,
)


def _tc2_body(sa, sb, y1, disb, b1, w2, y2):
    dis = disb[...]
    pre = dis * (sa[...] + sb[...] + y1[...]) + b1[...]
    a = jnp.where(pre > 0, pre, jnp.exp(pre) - 1.0)
    y2[...] = jnp.dot(a, w2[...], preferred_element_type=jnp.float32) * dis


_tc2 = pl.pallas_call(
    _tc2_body,
    grid=(NB,),
    in_specs=[
        pl.BlockSpec((R, D), lambda i: (i, 0)),
        pl.BlockSpec((R, D), lambda i: (i, 0)),
        pl.BlockSpec((R, D), lambda i: (i, 0)),
        pl.BlockSpec((R, D), lambda i: (i, 0)),
        pl.BlockSpec((1, D), lambda i: (0, 0)),
        pl.BlockSpec((D, D), lambda i: (0, 0)),
    ],
    out_specs=pl.BlockSpec((R, D), lambda i: (i, 0)),
    out_shape=jax.ShapeDtypeStruct((NP, D), jnp.float32),
)


def _tc3_body(sa, sb, y2, disb, b2, batr, stats, f1wa, f1wb, f1b, f2w, f2b,
              out, pool_acc, cnt_acc):
    i = pl.program_id(0)
    dis = disb[...]
    out2 = dis * (sa[...] + sb[...] + y2[...]) + b2[...]
    bat = jnp.reshape(batr[...], (1, R))
    onehot = (
        lax.broadcasted_iota(jnp.int32, (G, R), 0) == jnp.broadcast_to(bat, (G, R))
    ).astype(jnp.float32)
    psum = jnp.dot(onehot, out2, preferred_element_type=jnp.float32)
    csum = jnp.sum(onehot, axis=1, keepdims=True)

    @pl.when(i == 0)
    def _init():
        pool_acc[...] = psum
        cnt_acc[...] = csum

    @pl.when(i > 0)
    def _acc():
        pool_acc[...] += psum
        cnt_acc[...] += csum

    @pl.when(i == NB - 1)
    def _fin():
        pooled = pool_acc[...] / jnp.maximum(cnt_acc[...], 1.0)
        z = (
            jnp.dot(pooled, f1wa[...], preferred_element_type=jnp.float32)
            + jnp.dot(stats[...], f1wb[...], preferred_element_type=jnp.float32)
            + f1b[...]
        )
        z = jnp.maximum(z, 0.0)
        z = jnp.dot(z, f2w[...], preferred_element_type=jnp.float32) + f2b[...]
        m = jnp.max(z, axis=1, keepdims=True)
        e = jnp.exp(z - m)
        out[...] = z - m - jnp.log(jnp.sum(e, axis=1, keepdims=True))


_tc3 = pl.pallas_call(
    _tc3_body,
    grid=(NB,),
    in_specs=[
        pl.BlockSpec((R, D), lambda i: (i, 0)),
        pl.BlockSpec((R, D), lambda i: (i, 0)),
        pl.BlockSpec((R, D), lambda i: (i, 0)),
        pl.BlockSpec((R, D), lambda i: (i, 0)),
        pl.BlockSpec((1, D), lambda i: (0, 0)),
        pl.BlockSpec((1, 1, R), lambda i: (i, 0, 0)),
        pl.BlockSpec((G, 10), lambda i: (0, 0)),
        pl.BlockSpec((D, 20), lambda i: (0, 0)),
        pl.BlockSpec((10, 20), lambda i: (0, 0)),
        pl.BlockSpec((1, 20), lambda i: (0, 0)),
        pl.BlockSpec((20, 5), lambda i: (0, 0)),
        pl.BlockSpec((1, 5), lambda i: (0, 0)),
    ],
    out_specs=pl.BlockSpec((G, 5), lambda i: (0, 0)),
    out_shape=jax.ShapeDtypeStruct((G, 5), jnp.float32),
    scratch_shapes=[
        pltpu.VMEM((G, D), jnp.float32),
        pltpu.VMEM((G, 1), jnp.float32),
    ],
)


def kernel(x, edge_index, batch, eig, stats, W1, b1, W2, b2,
           fc1_W, fc1_b, fc2_W, fc2_b):
    del eig
    src = edge_index[0]
    dst = edge_index[1]
    pad = jnp.full((EPAD - E,), N, jnp.int32)
    srcp = jnp.concatenate([src, pad])
    dstp = jnp.concatenate([dst, pad])
    comb = jnp.bitwise_or(srcp, dstp << 16).reshape(TB, K)
    comb = jnp.zeros((TBP, K), jnp.int32).at[:TB].set(comb)
    xp = jnp.zeros((NP, D), jnp.float32).at[:N].set(x)

    deg2 = _deg_kernel(dst)
    deg_a = deg2[0].reshape(NP, 1)
    deg_b = deg2[1].reshape(NP, 1)
    y1, dis_b = _tc1(deg_a, deg_b, xp, W1)
    seg1 = _seg_kernel(comb, y1)
    y2 = _tc2(seg1[0], seg1[1], y1, dis_b, b1.reshape(1, D), W2)
    seg2 = _seg_kernel(comb, y2)
    batp = jnp.full((NP,), G, jnp.int32).at[:N].set(batch).reshape(NB, 1, R)
    return _tc3(seg2[0], seg2[1], y2, dis_b, b2.reshape(1, D),
                batp, stats, fc1_W[:D], fc1_W[D:], fc1_b.reshape(1, 20),
                fc2_W, fc2_b.reshape(1, 5))


# speedup vs baseline: 1.1753x; 1.1750x over previous
import functools

import jax
import jax.numpy as jnp
from jax import lax
from jax.experimental import pallas as pl
from jax.experimental.pallas import tpu as pltpu
from jax.experimental.pallas import tpu_sc as plsc

N = 10000
E = 320000
D = 128
G = 64
NC = 2
NS = 16
NW = NC * NS
NP = 10240
RT = NP // NS
K = 112
EW = E // NW
BT = 184
TB = NS * BT
TBP = 3072
B0 = 152
B1 = BT - B0
EPAD = TB * K
R = 1280
NB = NP // R


def _mesh():
    return plsc.VectorSubcoreMesh(
        core_axis_name="c", subcore_axis_name="s", num_cores=NC, num_subcores=NS
    )


@functools.partial(
    pl.kernel,
    out_type=jax.ShapeDtypeStruct((NC, NP), jnp.float32),
    mesh=_mesh(),
    scratch_types=[
        pltpu.VMEM((EW,), jnp.int32),
        pltpu.VMEM((NP,), jnp.float32),
        pltpu.VMEM((RT,), jnp.float32),
        pltpu.VMEM((RT,), jnp.float32),
        pltpu.VMEM_SHARED((NS, NP), jnp.float32),
    ],
    compiler_params=pltpu.CompilerParams(needs_layout_passes=False),
)
def _deg_kernel(dst_hbm, out_hbm, dst_v, hist_v, col_v, tmp_v, stage_sh):
    c = lax.axis_index("c")
    s = lax.axis_index("s")
    w = c * NS + s
    pltpu.sync_copy(dst_hbm.at[pl.ds(w * EW, EW)], dst_v)
    zero16 = jnp.zeros((16,), jnp.float32)

    def zb(i, _):
        hist_v[pl.ds(i * 16, 16)] = zero16
        return 0

    lax.fori_loop(0, NP // 16, zb, 0, unroll=4)
    one16 = jnp.ones((16,), jnp.float32)

    def sb(i, _):
        plsc.addupdate_scatter(hist_v, [dst_v[pl.ds(i * 16, 16)]], one16)
        return 0

    lax.fori_loop(0, EW // 16, sb, 0, unroll=4)
    pltpu.sync_copy(hist_v, stage_sh.at[s])
    plsc.subcore_barrier()
    base = s * RT
    pltpu.sync_copy(stage_sh.at[0, pl.ds(base, RT)], col_v)
    for t in range(1, NS):
        pltpu.sync_copy(stage_sh.at[t, pl.ds(base, RT)], tmp_v)

        def ab(j, _):
            col_v[pl.ds(j * 16, 16)] = col_v[pl.ds(j * 16, 16)] + tmp_v[pl.ds(j * 16, 16)]
            return 0

        lax.fori_loop(0, RT // 16, ab, 0, unroll=4)
    pltpu.sync_copy(col_v, out_hbm.at[c, pl.ds(base, RT)])


@functools.partial(
    pl.kernel,
    out_type=jax.ShapeDtypeStruct((NC, NP, D), jnp.float32),
    mesh=_mesh(),
    scratch_types=[
        pltpu.VMEM((B0, K), jnp.int32),
        [pltpu.VMEM((K,), jnp.int32) for _ in range(2)],
        [pltpu.VMEM((K,), jnp.int32) for _ in range(2)],
        [pltpu.VMEM((K, D), jnp.float32) for _ in range(2)],
        pltpu.VMEM_SHARED((NP, D), jnp.float32),
        pltpu.SemaphoreType.DMA,
        [pltpu.SemaphoreType.DMA for _ in range(2)],
    ],
    compiler_params=pltpu.CompilerParams(needs_layout_passes=False),
)
def _seg_kernel(comb_hbm, y_hbm, out_hbm,
                comb_v, srcb, dstb, rows, acc_sh, csem, gsems):
    c = lax.axis_index("c")
    s = lax.axis_index("s")
    base = s * RT
    start = s * BT + c * B0
    nb = jnp.where(c == 0, B0, B1)
    pltpu.async_copy(comb_hbm.at[pl.ds(start, B0)], comb_v, csem)
    zero16 = jnp.zeros((16,), jnp.float32)

    def zb(i, _):
        for j in range(D // 16):
            rows[0][i, pl.ds(j * 16, 16)] = zero16
        return 0

    lax.fori_loop(0, K, zb, 0, unroll=4)
    for j in range(RT // K):
        pltpu.sync_copy(rows[0], acc_sh.at[pl.ds(base + j * K, K)])
    rem = RT - (RT // K) * K
    if rem:
        pltpu.sync_copy(
            rows[0].at[pl.ds(0, rem)], acc_sh.at[pl.ds(base + (RT // K) * K, rem)]
        )
    plsc.subcore_barrier()
    pltpu.make_async_copy(comb_hbm.at[pl.ds(start, B0)], comb_v, csem).wait()

    def unpack(r, slot):
        for g in range(K // 16):
            vv = comb_v[r, pl.ds(g * 16, 16)]
            srcb[slot][pl.ds(g * 16, 16)] = jnp.bitwise_and(vv, 0xFFFF)
            dstb[slot][pl.ds(g * 16, 16)] = jax.lax.shift_right_logical(vv, 16)

    unpack(0, 0)
    pltpu.async_copy(y_hbm.at[srcb[0]], rows[0], gsems[0])

    def run(nbs):
        def eb(j, _):
            for b in range(2):
                i = j * 2 + b
                pltpu.make_async_copy(y_hbm.at[srcb[b]], rows[b], gsems[b]).wait()

                @pl.when(i + 1 < nbs)
                def _g():
                    unpack(i + 1, 1 - b)
                    pltpu.async_copy(y_hbm.at[srcb[1 - b]], rows[1 - b], gsems[1 - b])

                pltpu.sync_copy(rows[b], acc_sh.at[dstb[b]], add=True)
            return 0

        lax.fori_loop(0, nbs // 2, eb, 0)

    @pl.when(c == 0)
    def _run0():
        run(B0)

    @pl.when(c == 1)
    def _run1():
        run(B1)

    plsc.subcore_barrier()
    pltpu.sync_copy(acc_sh.at[pl.ds(base, RT)], out_hbm.at[c, pl.ds(base, RT)])


def _tc1_body(dega, degb, x, w1, y, disb):
    i = pl.program_id(0)
    rows = lax.broadcasted_iota(jnp.int32, (R, 1), 0) + i * R
    m = (rows < N).astype(jnp.float32)
    dis = m * lax.rsqrt(dega[...] + degb[...] + 1.0)
    h = jnp.dot(x[...], w1[...], preferred_element_type=jnp.float32)
    y[...] = h * dis
    disb[...] = jnp.broadcast_to(dis, (R, D))


_tc1 = pl.pallas_call(
    _tc1_body,
    grid=(NB,),
    in_specs=[
        pl.BlockSpec((R, 1), lambda i: (i, 0)),
        pl.BlockSpec((R, 1), lambda i: (i, 0)),
        pl.BlockSpec((R, D), lambda i: (i, 0)),
        pl.BlockSpec((D, D), lambda i: (0, 0)),
    ],
    out_specs=[pl.BlockSpec((R, D), lambda i: (i, 0))] * 2,
    out_shape=[jax.ShapeDtypeStruct((NP, D), jnp.float32)] * 2,
)


def _tc2_body(sa, sb, y1, disb, b1, w2, y2):
    dis = disb[...]
    pre = dis * (sa[...] + sb[...] + y1[...]) + b1[...]
    a = jnp.where(pre > 0, pre, jnp.exp(pre) - 1.0)
    y2[...] = jnp.dot(a, w2[...], preferred_element_type=jnp.float32) * dis


_tc2 = pl.pallas_call(
    _tc2_body,
    grid=(NB,),
    in_specs=[
        pl.BlockSpec((R, D), lambda i: (i, 0)),
        pl.BlockSpec((R, D), lambda i: (i, 0)),
        pl.BlockSpec((R, D), lambda i: (i, 0)),
        pl.BlockSpec((R, D), lambda i: (i, 0)),
        pl.BlockSpec((1, D), lambda i: (0, 0)),
        pl.BlockSpec((D, D), lambda i: (0, 0)),
    ],
    out_specs=pl.BlockSpec((R, D), lambda i: (i, 0)),
    out_shape=jax.ShapeDtypeStruct((NP, D), jnp.float32),
)


def _tc3_body(sa, sb, y2, disb, b2, batr, stats, f1wa, f1wb, f1b, f2w, f2b,
              out, pool_acc, cnt_acc):
    i = pl.program_id(0)
    dis = disb[...]
    out2 = dis * (sa[...] + sb[...] + y2[...]) + b2[...]
    bat = jnp.reshape(batr[...], (1, R))
    onehot = (
        lax.broadcasted_iota(jnp.int32, (G, R), 0) == jnp.broadcast_to(bat, (G, R))
    ).astype(jnp.float32)
    psum = jnp.dot(onehot, out2, preferred_element_type=jnp.float32)
    csum = jnp.sum(onehot, axis=1, keepdims=True)

    @pl.when(i == 0)
    def _init():
        pool_acc[...] = psum
        cnt_acc[...] = csum

    @pl.when(i > 0)
    def _acc():
        pool_acc[...] += psum
        cnt_acc[...] += csum

    @pl.when(i == NB - 1)
    def _fin():
        pooled = pool_acc[...] / jnp.maximum(cnt_acc[...], 1.0)
        z = (
            jnp.dot(pooled, f1wa[...], preferred_element_type=jnp.float32)
            + jnp.dot(stats[...], f1wb[...], preferred_element_type=jnp.float32)
            + f1b[...]
        )
        z = jnp.maximum(z, 0.0)
        z = jnp.dot(z, f2w[...], preferred_element_type=jnp.float32) + f2b[...]
        m = jnp.max(z, axis=1, keepdims=True)
        e = jnp.exp(z - m)
        out[...] = z - m - jnp.log(jnp.sum(e, axis=1, keepdims=True))


_tc3 = pl.pallas_call(
    _tc3_body,
    grid=(NB,),
    in_specs=[
        pl.BlockSpec((R, D), lambda i: (i, 0)),
        pl.BlockSpec((R, D), lambda i: (i, 0)),
        pl.BlockSpec((R, D), lambda i: (i, 0)),
        pl.BlockSpec((R, D), lambda i: (i, 0)),
        pl.BlockSpec((1, D), lambda i: (0, 0)),
        pl.BlockSpec((1, 1, R), lambda i: (i, 0, 0)),
        pl.BlockSpec((G, 10), lambda i: (0, 0)),
        pl.BlockSpec((D, 20), lambda i: (0, 0)),
        pl.BlockSpec((10, 20), lambda i: (0, 0)),
        pl.BlockSpec((1, 20), lambda i: (0, 0)),
        pl.BlockSpec((20, 5), lambda i: (0, 0)),
        pl.BlockSpec((1, 5), lambda i: (0, 0)),
    ],
    out_specs=pl.BlockSpec((G, 5), lambda i: (0, 0)),
    out_shape=jax.ShapeDtypeStruct((G, 5), jnp.float32),
    scratch_shapes=[
        pltpu.VMEM((G, D), jnp.float32),
        pltpu.VMEM((G, 1), jnp.float32),
    ],
)


def kernel(x, edge_index, batch, eig, stats, W1, b1, W2, b2,
           fc1_W, fc1_b, fc2_W, fc2_b):
    del eig
    src = edge_index[0]
    dst = edge_index[1]
    pad = jnp.full((EPAD - E,), N, jnp.int32)
    srcp = jnp.concatenate([src, pad])
    dstp = jnp.concatenate([dst, pad])
    comb = jnp.bitwise_or(srcp, dstp << 16).reshape(TB, K)
    comb = jnp.zeros((TBP, K), jnp.int32).at[:TB].set(comb)
    xp = jnp.zeros((NP, D), jnp.float32).at[:N].set(x)

    deg2 = _deg_kernel(dst)
    deg_a = deg2[0].reshape(NP, 1)
    deg_b = deg2[1].reshape(NP, 1)
    y1, dis_b = _tc1(deg_a, deg_b, xp, W1)
    seg1 = _seg_kernel(comb, y1)
    y2 = _tc2(seg1[0], seg1[1], y1, dis_b, b1.reshape(1, D), W2)
    seg2 = _seg_kernel(comb, y2)
    batp = jnp.full((NP,), G, jnp.int32).at[:N].set(batch).reshape(NB, 1, R)
    return _tc3(seg2[0], seg2[1], y2, dis_b, b2.reshape(1, D),
                batp, stats, fc1_W[:D], fc1_W[D:], fc1_b.reshape(1, 20),
                fc2_W, fc2_b.reshape(1, 5))


# speedup vs baseline: 1.1761x; 1.0007x over previous
import functools

import jax
import jax.numpy as jnp
from jax import lax
from jax.experimental import pallas as pl
from jax.experimental.pallas import tpu as pltpu
from jax.experimental.pallas import tpu_sc as plsc

N = 10000
E = 320000
D = 128
G = 64
NC = 2
NS = 16
NW = NC * NS
NP = 10240
RT = NP // NS
K = 112
EW = E // NW
BT = 184
TB = NS * BT
TBP = 3072
B0 = 152
B1 = BT - B0
EPAD = TB * K
R = 1280
NB = NP // R


def _mesh():
    return plsc.VectorSubcoreMesh(
        core_axis_name="c", subcore_axis_name="s", num_cores=NC, num_subcores=NS
    )


@functools.partial(
    pl.kernel,
    out_type=jax.ShapeDtypeStruct((NC, NP), jnp.float32),
    mesh=_mesh(),
    scratch_types=[
        pltpu.VMEM((EW,), jnp.int32),
        pltpu.VMEM((NP,), jnp.float32),
        pltpu.VMEM((RT,), jnp.float32),
        pltpu.VMEM((RT,), jnp.float32),
        pltpu.VMEM_SHARED((NS, NP), jnp.float32),
    ],
    compiler_params=pltpu.CompilerParams(needs_layout_passes=False),
)
def _deg_kernel(dst_hbm, out_hbm, dst_v, hist_v, col_v, tmp_v, stage_sh):
    c = lax.axis_index("c")
    s = lax.axis_index("s")
    w = c * NS + s
    pltpu.sync_copy(dst_hbm.at[pl.ds(w * EW, EW)], dst_v)
    zero16 = jnp.zeros((16,), jnp.float32)

    def zb(i, _):
        hist_v[pl.ds(i * 16, 16)] = zero16
        return 0

    lax.fori_loop(0, NP // 16, zb, 0, unroll=4)
    one16 = jnp.ones((16,), jnp.float32)

    def sb(i, _):
        plsc.addupdate_scatter(hist_v, [dst_v[pl.ds(i * 16, 16)]], one16)
        return 0

    lax.fori_loop(0, EW // 16, sb, 0, unroll=4)
    pltpu.sync_copy(hist_v, stage_sh.at[s])
    plsc.subcore_barrier()
    base = s * RT
    pltpu.sync_copy(stage_sh.at[0, pl.ds(base, RT)], col_v)
    for t in range(1, NS):
        pltpu.sync_copy(stage_sh.at[t, pl.ds(base, RT)], tmp_v)

        def ab(j, _):
            col_v[pl.ds(j * 16, 16)] = col_v[pl.ds(j * 16, 16)] + tmp_v[pl.ds(j * 16, 16)]
            return 0

        lax.fori_loop(0, RT // 16, ab, 0, unroll=4)
    pltpu.sync_copy(col_v, out_hbm.at[c, pl.ds(base, RT)])


@functools.partial(
    pl.kernel,
    out_type=jax.ShapeDtypeStruct((NC, NP, D), jnp.float32),
    mesh=_mesh(),
    scratch_types=[
        pltpu.VMEM((B0, K), jnp.int32),
        [pltpu.VMEM((K,), jnp.int32) for _ in range(2)],
        pltpu.VMEM((2, K), jnp.int32),
        [pltpu.VMEM((K, D), jnp.float32) for _ in range(2)],
        pltpu.VMEM_SHARED((NP, D), jnp.float32),
        pltpu.SemaphoreType.DMA,
        [pltpu.SemaphoreType.DMA for _ in range(2)],
    ],
    compiler_params=pltpu.CompilerParams(needs_layout_passes=False),
)
def _seg_kernel(comb_hbm, y_hbm, out_hbm,
                comb_v, srcb, dstb, rows, acc_sh, csem, gsems):
    c = lax.axis_index("c")
    s = lax.axis_index("s")
    base = s * RT
    start = s * BT + c * B0
    nb = jnp.where(c == 0, B0, B1)
    pltpu.async_copy(comb_hbm.at[pl.ds(start, B0)], comb_v, csem)
    zero16 = jnp.zeros((16,), jnp.float32)

    def zb(i, _):
        for j in range(D // 16):
            rows[0][i, pl.ds(j * 16, 16)] = zero16
        return 0

    lax.fori_loop(0, K, zb, 0, unroll=4)
    for j in range(RT // K):
        pltpu.sync_copy(rows[0], acc_sh.at[pl.ds(base + j * K, K)])
    rem = RT - (RT // K) * K
    if rem:
        pltpu.sync_copy(
            rows[0].at[pl.ds(0, rem)], acc_sh.at[pl.ds(base + (RT // K) * K, rem)]
        )
    plsc.subcore_barrier()
    pltpu.make_async_copy(comb_hbm.at[pl.ds(start, B0)], comb_v, csem).wait()

    def unpack(r, slot):
        for g in range(K // 16):
            vv = comb_v[r, pl.ds(g * 16, 16)]
            srcb[slot][pl.ds(g * 16, 16)] = jnp.bitwise_and(vv, 0xFFFF)
            dstb[slot, pl.ds(g * 16, 16)] = jax.lax.shift_right_logical(vv, 16)

    unpack(0, 0)
    pltpu.async_copy(y_hbm.at[srcb[0]], rows[0], gsems[0])

    def run(nbs):
        def eb(j, _):
            for b in range(2):
                i = j * 2 + b
                pltpu.make_async_copy(y_hbm.at[srcb[b]], rows[b], gsems[b]).wait()

                @pl.when(i + 1 < nbs)
                def _g():
                    unpack(i + 1, 1 - b)
                    pltpu.async_copy(y_hbm.at[srcb[1 - b]], rows[1 - b], gsems[1 - b])

                pltpu.sync_copy(rows[b], acc_sh.at[dstb.at[b]], add=True)
            return 0

        lax.fori_loop(0, nbs // 2, eb, 0)

    @pl.when(c == 0)
    def _run0():
        run(B0)

    @pl.when(c == 1)
    def _run1():
        run(B1)

    plsc.subcore_barrier()
    pltpu.sync_copy(acc_sh.at[pl.ds(base, RT)], out_hbm.at[c, pl.ds(base, RT)])


def _tc1_body(dega, degb, x, w1, y, disb):
    i = pl.program_id(0)
    rows = lax.broadcasted_iota(jnp.int32, (R, 1), 0) + i * R
    m = (rows < N).astype(jnp.float32)
    dis = m * lax.rsqrt(dega[...] + degb[...] + 1.0)
    h = jnp.dot(x[...], w1[...], preferred_element_type=jnp.float32)
    y[...] = h * dis
    disb[...] = jnp.broadcast_to(dis, (R, D))


_tc1 = pl.pallas_call(
    _tc1_body,
    grid=(NB,),
    in_specs=[
        pl.BlockSpec((R, 1), lambda i: (i, 0)),
        pl.BlockSpec((R, 1), lambda i: (i, 0)),
        pl.BlockSpec((R, D), lambda i: (i, 0)),
        pl.BlockSpec((D, D), lambda i: (0, 0)),
    ],
    out_specs=[pl.BlockSpec((R, D), lambda i: (i, 0))] * 2,
    out_shape=[jax.ShapeDtypeStruct((NP, D), jnp.float32)] * 2,
)


def _tc2_body(sa, sb, y1, disb, b1, w2, y2):
    dis = disb[...]
    pre = dis * (sa[...] + sb[...] + y1[...]) + b1[...]
    a = jnp.where(pre > 0, pre, jnp.exp(pre) - 1.0)
    y2[...] = jnp.dot(a, w2[...], preferred_element_type=jnp.float32) * dis


_tc2 = pl.pallas_call(
    _tc2_body,
    grid=(NB,),
    in_specs=[
        pl.BlockSpec((R, D), lambda i: (i, 0)),
        pl.BlockSpec((R, D), lambda i: (i, 0)),
        pl.BlockSpec((R, D), lambda i: (i, 0)),
        pl.BlockSpec((R, D), lambda i: (i, 0)),
        pl.BlockSpec((1, D), lambda i: (0, 0)),
        pl.BlockSpec((D, D), lambda i: (0, 0)),
    ],
    out_specs=pl.BlockSpec((R, D), lambda i: (i, 0)),
    out_shape=jax.ShapeDtypeStruct((NP, D), jnp.float32),
)


def _tc3_body(sa, sb, y2, disb, b2, batr, stats, f1wa, f1wb, f1b, f2w, f2b,
              out, pool_acc, cnt_acc):
    i = pl.program_id(0)
    dis = disb[...]
    out2 = dis * (sa[...] + sb[...] + y2[...]) + b2[...]
    bat = jnp.reshape(batr[...], (1, R))
    onehot = (
        lax.broadcasted_iota(jnp.int32, (G, R), 0) == jnp.broadcast_to(bat, (G, R))
    ).astype(jnp.float32)
    psum = jnp.dot(onehot, out2, preferred_element_type=jnp.float32)
    csum = jnp.sum(onehot, axis=1, keepdims=True)

    @pl.when(i == 0)
    def _init():
        pool_acc[...] = psum
        cnt_acc[...] = csum

    @pl.when(i > 0)
    def _acc():
        pool_acc[...] += psum
        cnt_acc[...] += csum

    @pl.when(i == NB - 1)
    def _fin():
        pooled = pool_acc[...] / jnp.maximum(cnt_acc[...], 1.0)
        z = (
            jnp.dot(pooled, f1wa[...], preferred_element_type=jnp.float32)
            + jnp.dot(stats[...], f1wb[...], preferred_element_type=jnp.float32)
            + f1b[...]
        )
        z = jnp.maximum(z, 0.0)
        z = jnp.dot(z, f2w[...], preferred_element_type=jnp.float32) + f2b[...]
        m = jnp.max(z, axis=1, keepdims=True)
        e = jnp.exp(z - m)
        out[...] = z - m - jnp.log(jnp.sum(e, axis=1, keepdims=True))


_tc3 = pl.pallas_call(
    _tc3_body,
    grid=(NB,),
    in_specs=[
        pl.BlockSpec((R, D), lambda i: (i, 0)),
        pl.BlockSpec((R, D), lambda i: (i, 0)),
        pl.BlockSpec((R, D), lambda i: (i, 0)),
        pl.BlockSpec((R, D), lambda i: (i, 0)),
        pl.BlockSpec((1, D), lambda i: (0, 0)),
        pl.BlockSpec((1, 1, R), lambda i: (i, 0, 0)),
        pl.BlockSpec((G, 10), lambda i: (0, 0)),
        pl.BlockSpec((D, 20), lambda i: (0, 0)),
        pl.BlockSpec((10, 20), lambda i: (0, 0)),
        pl.BlockSpec((1, 20), lambda i: (0, 0)),
        pl.BlockSpec((20, 5), lambda i: (0, 0)),
        pl.BlockSpec((1, 5), lambda i: (0, 0)),
    ],
    out_specs=pl.BlockSpec((G, 5), lambda i: (0, 0)),
    out_shape=jax.ShapeDtypeStruct((G, 5), jnp.float32),
    scratch_shapes=[
        pltpu.VMEM((G, D), jnp.float32),
        pltpu.VMEM((G, 1), jnp.float32),
    ],
)


def kernel(x, edge_index, batch, eig, stats, W1, b1, W2, b2,
           fc1_W, fc1_b, fc2_W, fc2_b):
    del eig
    src = edge_index[0]
    dst = edge_index[1]
    pad = jnp.full((EPAD - E,), N, jnp.int32)
    srcp = jnp.concatenate([src, pad])
    dstp = jnp.concatenate([dst, pad])
    comb = jnp.bitwise_or(srcp, dstp << 16).reshape(TB, K)
    comb = jnp.zeros((TBP, K), jnp.int32).at[:TB].set(comb)
    xp = jnp.zeros((NP, D), jnp.float32).at[:N].set(x)

    deg2 = _deg_kernel(dst)
    deg_a = deg2[0].reshape(NP, 1)
    deg_b = deg2[1].reshape(NP, 1)
    y1, dis_b = _tc1(deg_a, deg_b, xp, W1)
    seg1 = _seg_kernel(comb, y1)
    y2 = _tc2(seg1[0], seg1[1], y1, dis_b, b1.reshape(1, D), W2)
    seg2 = _seg_kernel(comb, y2)
    batp = jnp.full((NP,), G, jnp.int32).at[:N].set(batch).reshape(NB, 1, R)
    return _tc3(seg2[0], seg2[1], y2, dis_b, b2.reshape(1, D),
                batp, stats, fc1_W[:D], fc1_W[D:], fc1_b.reshape(1, 20),
                fc2_W, fc2_b.reshape(1, 5))


# speedup vs baseline: 2.8266x; 2.4034x over previous
import functools

import jax
import jax.numpy as jnp
from jax import lax
from jax.experimental import pallas as pl
from jax.experimental.pallas import tpu as pltpu
from jax.experimental.pallas import tpu_sc as plsc

N = 10000
E = 320000
D = 128
G = 64
NC = 2
NS = 16
NW = NC * NS
NP = 10240
RT = NP // NS
K = 128
EW = E // NW
BT = 160
TB = NS * BT
TBP = 2688
B0 = 120
B1 = BT - B0
EPAD = TB * K
R = 1280
NB = NP // R


def _mesh():
    return plsc.VectorSubcoreMesh(
        core_axis_name="c", subcore_axis_name="s", num_cores=NC, num_subcores=NS
    )


@functools.partial(
    pl.kernel,
    out_type=jax.ShapeDtypeStruct((NC, NP), jnp.float32),
    mesh=_mesh(),
    scratch_types=[
        pltpu.VMEM((EW,), jnp.int32),
        pltpu.VMEM((NP,), jnp.float32),
        pltpu.VMEM((RT,), jnp.float32),
        pltpu.VMEM((RT,), jnp.float32),
        pltpu.VMEM_SHARED((NS, NP), jnp.float32),
    ],
    compiler_params=pltpu.CompilerParams(needs_layout_passes=False),
)
def _deg_kernel(dst_hbm, out_hbm, dst_v, hist_v, col_v, tmp_v, stage_sh):
    c = lax.axis_index("c")
    s = lax.axis_index("s")
    w = c * NS + s
    pltpu.sync_copy(dst_hbm.at[pl.ds(w * EW, EW)], dst_v)
    zero16 = jnp.zeros((16,), jnp.float32)

    def zb(i, _):
        hist_v[pl.ds(i * 16, 16)] = zero16
        return 0

    lax.fori_loop(0, NP // 16, zb, 0, unroll=4)
    one16 = jnp.ones((16,), jnp.float32)

    def sb(i, _):
        plsc.addupdate_scatter(hist_v, [dst_v[pl.ds(i * 16, 16)]], one16)
        return 0

    lax.fori_loop(0, EW // 16, sb, 0, unroll=4)
    pltpu.sync_copy(hist_v, stage_sh.at[s])
    plsc.subcore_barrier()
    base = s * RT
    pltpu.sync_copy(stage_sh.at[0, pl.ds(base, RT)], col_v)
    for t in range(1, NS):
        pltpu.sync_copy(stage_sh.at[t, pl.ds(base, RT)], tmp_v)

        def ab(j, _):
            col_v[pl.ds(j * 16, 16)] = col_v[pl.ds(j * 16, 16)] + tmp_v[pl.ds(j * 16, 16)]
            return 0

        lax.fori_loop(0, RT // 16, ab, 0, unroll=4)
    pltpu.sync_copy(col_v, out_hbm.at[c, pl.ds(base, RT)])


@functools.partial(
    pl.kernel,
    out_type=jax.ShapeDtypeStruct((NC, NP, D), jnp.float32),
    mesh=_mesh(),
    scratch_types=[
        [pltpu.VMEM((K,), jnp.int32) for _ in range(2)],
        pltpu.VMEM((B0, K), jnp.int32),
        [pltpu.VMEM((K, D), jnp.float32) for _ in range(2)],
        pltpu.VMEM_SHARED((NP, D), jnp.float32),
        [pltpu.SemaphoreType.DMA for _ in range(2)],
        [pltpu.SemaphoreType.DMA for _ in range(2)],
    ],
    compiler_params=pltpu.CompilerParams(needs_layout_passes=False),
)
def _seg_kernel(src2_hbm, dst2_hbm, y_hbm, out_hbm,
                srcb, dstb_v, rows, acc_sh, isems, gsems):
    c = lax.axis_index("c")
    s = lax.axis_index("s")
    base = s * RT
    start = s * BT + c * B0
    pltpu.sync_copy(dst2_hbm.at[pl.ds(start, B0)], dstb_v)
    zero16 = jnp.zeros((16,), jnp.float32)

    def zb(i, _):
        for j in range(D // 16):
            rows[0][i, pl.ds(j * 16, 16)] = zero16
        return 0

    lax.fori_loop(0, K, zb, 0, unroll=4)
    for j in range(RT // K):
        pltpu.sync_copy(rows[0], acc_sh.at[pl.ds(base + j * K, K)])
    plsc.subcore_barrier()
    pltpu.async_copy(src2_hbm.at[start], srcb[0], isems[0])
    pltpu.async_copy(src2_hbm.at[start + 1], srcb[1], isems[1])
    pltpu.make_async_copy(src2_hbm.at[start], srcb[0], isems[0]).wait()
    pltpu.async_copy(y_hbm.at[srcb[0]], rows[0], gsems[0])

    def run(nbs):
        def eb(j, _):
            for b in range(2):
                i = j * 2 + b
                t = start + i
                pltpu.make_async_copy(y_hbm.at[srcb[b]], rows[b], gsems[b]).wait()

                @pl.when(i + 1 < nbs)
                def _g():
                    pltpu.make_async_copy(
                        src2_hbm.at[t + 1], srcb[1 - b], isems[1 - b]
                    ).wait()
                    pltpu.async_copy(y_hbm.at[srcb[1 - b]], rows[1 - b], gsems[1 - b])

                @pl.when(i + 2 < nbs)
                def _s2():
                    pltpu.async_copy(src2_hbm.at[t + 2], srcb[b], isems[b])

                pltpu.sync_copy(rows[b], acc_sh.at[dstb_v.at[i]], add=True)
            return 0

        lax.fori_loop(0, nbs // 2, eb, 0)

    @pl.when(c == 0)
    def _run0():
        run(B0)

    @pl.when(c == 1)
    def _run1():
        run(B1)

    plsc.subcore_barrier()
    pltpu.sync_copy(acc_sh.at[pl.ds(base, RT)], out_hbm.at[c, pl.ds(base, RT)])


def _tc1_body(dega, degb, x, w1, y, disb):
    i = pl.program_id(0)
    rows = lax.broadcasted_iota(jnp.int32, (R, 1), 0) + i * R
    m = (rows < N).astype(jnp.float32)
    dis = m * lax.rsqrt(dega[...] + degb[...] + 1.0)
    h = jnp.dot(x[...], w1[...], preferred_element_type=jnp.float32)
    y[...] = h * dis
    disb[...] = jnp.broadcast_to(dis, (R, D))


_tc1 = pl.pallas_call(
    _tc1_body,
    grid=(NB,),
    in_specs=[
        pl.BlockSpec((R, 1), lambda i: (i, 0)),
        pl.BlockSpec((R, 1), lambda i: (i, 0)),
        pl.BlockSpec((R, D), lambda i: (i, 0)),
        pl.BlockSpec((D, D), lambda i: (0, 0)),
    ],
    out_specs=[pl.BlockSpec((R, D), lambda i: (i, 0))] * 2,
    out_shape=[jax.ShapeDtypeStruct((NP, D), jnp.float32)] * 2,
)


def _tc2_body(sa, sb, y1, disb, b1, w2, y2):
    dis = disb[...]
    pre = dis * (sa[...] + sb[...] + y1[...]) + b1[...]
    a = jnp.where(pre > 0, pre, jnp.exp(pre) - 1.0)
    y2[...] = jnp.dot(a, w2[...], preferred_element_type=jnp.float32) * dis


_tc2 = pl.pallas_call(
    _tc2_body,
    grid=(NB,),
    in_specs=[
        pl.BlockSpec((R, D), lambda i: (i, 0)),
        pl.BlockSpec((R, D), lambda i: (i, 0)),
        pl.BlockSpec((R, D), lambda i: (i, 0)),
        pl.BlockSpec((R, D), lambda i: (i, 0)),
        pl.BlockSpec((1, D), lambda i: (0, 0)),
        pl.BlockSpec((D, D), lambda i: (0, 0)),
    ],
    out_specs=pl.BlockSpec((R, D), lambda i: (i, 0)),
    out_shape=jax.ShapeDtypeStruct((NP, D), jnp.float32),
)


def _tc3_body(sa, sb, y2, disb, b2, batr, stats, f1wa, f1wb, f1b, f2w, f2b,
              out, pool_acc, cnt_acc):
    i = pl.program_id(0)
    dis = disb[...]
    out2 = dis * (sa[...] + sb[...] + y2[...]) + b2[...]
    bat = jnp.reshape(batr[...], (1, R))
    onehot = (
        lax.broadcasted_iota(jnp.int32, (G, R), 0) == jnp.broadcast_to(bat, (G, R))
    ).astype(jnp.float32)
    psum = jnp.dot(onehot, out2, preferred_element_type=jnp.float32)
    csum = jnp.sum(onehot, axis=1, keepdims=True)

    @pl.when(i == 0)
    def _init():
        pool_acc[...] = psum
        cnt_acc[...] = csum

    @pl.when(i > 0)
    def _acc():
        pool_acc[...] += psum
        cnt_acc[...] += csum

    @pl.when(i == NB - 1)
    def _fin():
        pooled = pool_acc[...] / jnp.maximum(cnt_acc[...], 1.0)
        z = (
            jnp.dot(pooled, f1wa[...], preferred_element_type=jnp.float32)
            + jnp.dot(stats[...], f1wb[...], preferred_element_type=jnp.float32)
            + f1b[...]
        )
        z = jnp.maximum(z, 0.0)
        z = jnp.dot(z, f2w[...], preferred_element_type=jnp.float32) + f2b[...]
        m = jnp.max(z, axis=1, keepdims=True)
        e = jnp.exp(z - m)
        out[...] = z - m - jnp.log(jnp.sum(e, axis=1, keepdims=True))


_tc3 = pl.pallas_call(
    _tc3_body,
    grid=(NB,),
    in_specs=[
        pl.BlockSpec((R, D), lambda i: (i, 0)),
        pl.BlockSpec((R, D), lambda i: (i, 0)),
        pl.BlockSpec((R, D), lambda i: (i, 0)),
        pl.BlockSpec((R, D), lambda i: (i, 0)),
        pl.BlockSpec((1, D), lambda i: (0, 0)),
        pl.BlockSpec((1, 1, R), lambda i: (i, 0, 0)),
        pl.BlockSpec((G, 10), lambda i: (0, 0)),
        pl.BlockSpec((D, 20), lambda i: (0, 0)),
        pl.BlockSpec((10, 20), lambda i: (0, 0)),
        pl.BlockSpec((1, 20), lambda i: (0, 0)),
        pl.BlockSpec((20, 5), lambda i: (0, 0)),
        pl.BlockSpec((1, 5), lambda i: (0, 0)),
    ],
    out_specs=pl.BlockSpec((G, 5), lambda i: (0, 0)),
    out_shape=jax.ShapeDtypeStruct((G, 5), jnp.float32),
    scratch_shapes=[
        pltpu.VMEM((G, D), jnp.float32),
        pltpu.VMEM((G, 1), jnp.float32),
    ],
)


def kernel(x, edge_index, batch, eig, stats, W1, b1, W2, b2,
           fc1_W, fc1_b, fc2_W, fc2_b):
    del eig
    src = edge_index[0]
    dst = edge_index[1]
    pad = jnp.full((EPAD - E,), N, jnp.int32)
    srcp = jnp.concatenate([src, pad]).reshape(TB, K)
    dstp = jnp.concatenate([dst, pad]).reshape(TB, K)
    dstp = jnp.zeros((TBP, K), jnp.int32).at[:TB].set(dstp)
    xp = jnp.zeros((NP, D), jnp.float32).at[:N].set(x)

    deg2 = _deg_kernel(dst)
    deg_a = deg2[0].reshape(NP, 1)
    deg_b = deg2[1].reshape(NP, 1)
    y1, dis_b = _tc1(deg_a, deg_b, xp, W1)
    seg1 = _seg_kernel(srcp, dstp, y1)
    y2 = _tc2(seg1[0], seg1[1], y1, dis_b, b1.reshape(1, D), W2)
    seg2 = _seg_kernel(srcp, dstp, y2)
    batp = jnp.full((NP,), G, jnp.int32).at[:N].set(batch).reshape(NB, 1, R)
    return _tc3(seg2[0], seg2[1], y2, dis_b, b2.reshape(1, D),
                batp, stats, fc1_W[:D], fc1_W[D:], fc1_b.reshape(1, 20),
                fc2_W, fc2_b.reshape(1, 5))
